# Initial kernel scaffold; baseline (speedup 1.0000x reference)
import jax
import jax.numpy as jnp
from jax.experimental import pallas as pl


def kernel(h, incident_nodes, incident_edges, incident_values, degree_v_values, degree_e_values, sent_index, sent_values, layer, params):
    raise NotImplementedError("write your pallas kernel here")


# speedup vs baseline: 1.0192x; 1.0192x over previous
import jax
import jax.numpy as jnp
from jax.experimental import pallas as pl


def _gru_pallas(h, h_v, params):
    def body(h_ref, hv_ref, wiz, whz, wir, whr, win, whn, biz, bhz, bir, bhr,
             bin_, bhn, out_ref):
        hh = h_ref[...]
        hv = hv_ref[...]
        z = jax.nn.sigmoid(hh @ wiz[...] + biz[...] + hv @ whz[...] + bhz[...])
        r = jax.nn.sigmoid(hh @ wir[...] + bir[...] + hv @ whr[...] + bhr[...])
        n = jnp.tanh(hh @ win[...] + bin_[...] + (r * hv) @ whn[...] + bhn[...])
        out_ref[...] = (1.0 - z) * n + z * hv

    p = params
    return pl.pallas_call(
        body,
        out_shape=jax.ShapeDtypeStruct(h.shape, h.dtype),
    )(h, h_v, p['gru_Wiz'], p['gru_Whz'], p['gru_Wir'], p['gru_Whr'],
      p['gru_Win'], p['gru_Whn'], p['gru_biz'].reshape(1, -1),
      p['gru_bhz'].reshape(1, -1), p['gru_bir'].reshape(1, -1),
      p['gru_bhr'].reshape(1, -1), p['gru_bin'].reshape(1, -1),
      p['gru_bhn'].reshape(1, -1))


def _spmm(rows, cols, vals, m, X):
    return jnp.zeros((m, X.shape[1]), X.dtype).at[rows].add(vals[:, None] * X[cols])


def _mlp(x, W1, b1, W2, b2):
    return jnp.maximum(x @ W1 + b1, 0.0) @ W2 + b2


def _attn_mod(x, W1, b1, W2, b2):
    a = x @ W1 + b1
    a = jnp.where(a > 0, a, 0.01 * a)
    return a @ W2 + b2


def _bn(x, g, b):
    mu = jnp.mean(x, 0)
    var = jnp.var(x, 0)
    return (x - mu) / jnp.sqrt(var + 1e-5) * g + b


def kernel(h, incident_nodes, incident_edges, incident_values, degree_v_values,
           degree_e_values, sent_index, sent_values, layer, params):
    p = params
    N_v = h.shape[0]
    N_e = degree_e_values.shape[0]
    h_m = _mlp(h, p['mlp1_W1'], p['mlp1_b1'], p['mlp1_W2'], p['mlp1_b2'])
    h_t = _spmm(incident_edges, incident_nodes, incident_values, N_e, h_m)
    h_t = degree_e_values[:, None] * h_t
    ai = jnp.concatenate([h_m[incident_nodes], h_t[incident_edges]], axis=1)
    att = _attn_mod(ai, p['att1_W1'], p['att1_b1'], p['att1_W2'], p['att1_b2'])[:, 0]
    att = jnp.exp(att - jax.lax.stop_gradient(jnp.max(att)))
    pooled = _spmm(incident_edges, incident_nodes, att, N_e, h_m)
    row_sum = jnp.zeros((N_e, 1), h.dtype).at[incident_edges].add(att[:, None]) + 1e-20
    h_e = pooled / row_sum
    h_n = _spmm(sent_index[0], sent_index[1], sent_values, N_e, h_e)
    h_n = jnp.tanh(_bn(h_n, p['bn2_g'], p['bn2_b']))
    h_n = _mlp(h_n, p['mlp2_W1'], p['mlp2_b1'], p['mlp2_W2'], p['mlp2_b2'])
    h_t = _spmm(incident_nodes, incident_edges, incident_values, N_v, h_n)
    h_t = degree_v_values[:, None] * h_t
    ai = jnp.concatenate([h_n[incident_edges], h_t[incident_nodes]], axis=1)
    att = _attn_mod(ai, p['att2_W1'], p['att2_b1'], p['att2_W2'], p['att2_b2'])[:, 0]
    att = jnp.exp(att - jax.lax.stop_gradient(jnp.max(att)))
    pooled = _spmm(incident_nodes, incident_edges, att, N_v, h_n)
    row_sum = jnp.zeros((N_v, 1), h.dtype).at[incident_nodes].add(att[:, None]) + 1e-20
    h_v = pooled / row_sum
    h_v = jnp.tanh(_bn(h_v, p['bn1_g'], p['bn1_b']))
    return _gru_pallas(h, h_v, params)


# speedup vs baseline: 1.4242x; 1.3975x over previous
import functools

import jax
import jax.numpy as jnp
from jax import lax
from jax.experimental import pallas as pl
from jax.experimental.pallas import tpu as pltpu
from jax.experimental.pallas import tpu_sc as plsc

NC, NS, L = 2, 16, 16
NW = NC * NS
D = 128
DA = 256

N_V, N_E = 10000, 5000
NNZ, NNZS = 320000, 80000
PT = 10240
NNZ_PAD = NW * PT
PTS = 2560
NNZS_PAD = NW * PTS
NEP = 5120
NVP = 10240
T = 128

_MESH = plsc.VectorSubcoreMesh(core_axis_name="c", subcore_axis_name="s",
                               num_cores=NC, num_subcores=NS)

_GATHER_DN = lax.GatherDimensionNumbers(
    offset_dims=(), collapsed_slice_dims=(0,), start_index_map=(0,))


def _bcast_lane(vec, i):
    idx = jnp.full((L, 1), i, jnp.int32)
    return lax.gather(vec, idx, _GATHER_DN, (1,),
                      mode=lax.GatherScatterMode.PROMISE_IN_BOUNDS)


def _lane_reduce_sum(v):
    lane_iota = lax.iota(jnp.int32, L)
    for sh in (8, 4, 2, 1):
        idx = (lane_iota ^ sh).reshape(L, 1)
        v = v + lax.gather(v, idx, _GATHER_DN, (1,),
                           mode=lax.GatherScatterMode.PROMISE_IN_BOUNDS)
    return v


def _make_spmm(ndst_pad, per_tile, with_rowsum):
    nb = per_tile // T
    zrows = ndst_pad // NS

    outs = jax.ShapeDtypeStruct((NC, ndst_pad, D), jnp.float32)
    scratch = [
        pltpu.VMEM((T,), jnp.int32),
        pltpu.VMEM((T,), jnp.int32),
        pltpu.VMEM((T,), jnp.float32),
        pltpu.VMEM((T, D), jnp.float32),
        pltpu.VMEM_SHARED((ndst_pad, D), jnp.float32),
        pltpu.SemaphoreType.DMA,
    ]
    if with_rowsum:
        outs = (outs, jax.ShapeDtypeStruct((NW, ndst_pad), jnp.float32))
        scratch.append(pltpu.VMEM((ndst_pad,), jnp.float32))

    @functools.partial(pl.kernel, out_type=outs, mesh=_MESH,
                       scratch_types=scratch)
    def spmm(table, src_idx, dst_idx, w, *rest):
        if with_rowsum:
            out, out_rs, idxs_v, idxd_v, w_v, rows_v, acc, sem, rs_v = rest
        else:
            out, idxs_v, idxd_v, w_v, rows_v, acc, sem = rest
            out_rs = rs_v = None
        c = lax.axis_index("c")
        s = lax.axis_index("s")
        wid = c * NS + s
        lane_iota = lax.iota(jnp.int32, L)

        def zbody(i, carry):
            for cc in range(D // L):
                rows_v[i, pl.ds(cc * L, L)] = jnp.zeros((L,), jnp.float32)
            return carry
        lax.fori_loop(0, T, zbody, 0)
        if with_rowsum:
            def zrs(i, carry):
                rs_v[pl.ds(i * L, L)] = jnp.zeros((L,), jnp.float32)
                return carry
            lax.fori_loop(0, ndst_pad // L, zrs, 0)

        off = s * zrows
        done = 0
        while done < zrows:
            ch = min(zrows - done, T)
            pltpu.sync_copy(rows_v.at[pl.ds(0, ch)],
                            acc.at[pl.ds(off + done, ch)])
            done += ch
        plsc.subcore_barrier()

        base0 = wid * per_tile

        def batch(b, carry):
            base = base0 + b * T
            pltpu.sync_copy(src_idx.at[pl.ds(base, T)], idxs_v)
            pltpu.sync_copy(dst_idx.at[pl.ds(base, T)], idxd_v)
            pltpu.sync_copy(w.at[pl.ds(base, T)], w_v)
            pltpu.async_copy(table.at[idxs_v], rows_v, sem).wait()
            for g in range(T // L):
                wv = w_v[pl.ds(g * L, L)]
                if with_rowsum:
                    dv = idxd_v[pl.ds(g * L, L)]
                for i in range(L):
                    r = g * L + i
                    wb = _bcast_lane(wv, i)
                    for cc in range(D // L):
                        sl = pl.ds(cc * L, L)
                        rows_v[r, sl] = rows_v[r, sl] * wb
                    if with_rowsum:
                        e = dv[i]
                        win = rs_v[pl.ds(e, L)]
                        rs_v[pl.ds(e, L)] = win + jnp.where(
                            lane_iota == 0, wv[i], 0.0)
            pltpu.sync_copy(rows_v, acc.at[idxd_v], add=True)
            return carry
        lax.fori_loop(0, nb, batch, 0)
        plsc.subcore_barrier()

        done = 0
        while done < zrows:
            ch = min(zrows - done, T)
            pltpu.sync_copy(acc.at[pl.ds(off + done, ch)],
                            rows_v.at[pl.ds(0, ch)])
            pltpu.sync_copy(rows_v.at[pl.ds(0, ch)],
                            out.at[c, pl.ds(off + done, ch)])
            done += ch
        if with_rowsum:
            pltpu.sync_copy(rs_v, out_rs.at[wid])

    return spmm


def _make_att(per_tile):
    nb = per_tile // T

    @functools.partial(
        pl.kernel,
        out_type=jax.ShapeDtypeStruct((NW * per_tile,), jnp.float32),
        mesh=_MESH,
        scratch_types=[
            pltpu.VMEM((T,), jnp.int32),
            pltpu.VMEM((T,), jnp.int32),
            pltpu.VMEM((T, DA), jnp.float32),
            pltpu.VMEM((T, DA), jnp.float32),
            pltpu.VMEM((DA,), jnp.float32),
            pltpu.VMEM((T,), jnp.float32),
            pltpu.SemaphoreType.DMA,
            pltpu.SemaphoreType.DMA,
        ],
    )
    def att(tblA, idxA, tblB, idxB, w2, out, idxa_v, idxb_v, rowsA, rowsB,
            w2_v, att_v, semA, semB):
        c = lax.axis_index("c")
        s = lax.axis_index("s")
        wid = c * NS + s
        base0 = wid * per_tile
        pltpu.sync_copy(w2, w2_v)
        lane_iota = lax.iota(jnp.int32, L)

        def batch(b, carry):
            base = base0 + b * T
            pltpu.sync_copy(idxA.at[pl.ds(base, T)], idxa_v)
            pltpu.sync_copy(idxB.at[pl.ds(base, T)], idxb_v)
            cpa = pltpu.async_copy(tblA.at[idxa_v], rowsA, semA)
            cpb = pltpu.async_copy(tblB.at[idxb_v], rowsB, semB)
            cpa.wait()
            cpb.wait()
            w2c = [w2_v[pl.ds(cc * L, L)] for cc in range(DA // L)]

            def group(g, carry2):
                accv = jnp.zeros((L,), jnp.float32)
                for i in range(L):
                    r = g * L + i
                    acc = jnp.zeros((L,), jnp.float32)
                    for cc in range(DA // L):
                        sl = pl.ds(cc * L, L)
                        t = rowsA[r, sl] + rowsB[r, sl]
                        m = jnp.maximum(t, 0.01 * t)
                        acc = acc + m * w2c[cc]
                    red = _lane_reduce_sum(acc)
                    accv = jnp.where(lane_iota == i, red, accv)
                att_v[pl.ds(g * L, L)] = accv
                return carry2
            lax.fori_loop(0, T // L, group, 0)
            pltpu.sync_copy(att_v, out.at[pl.ds(base, T)])
            return carry
        lax.fori_loop(0, nb, batch, 0)

    return att


_spmm_e = _make_spmm(NEP, PT, False)
_spmm_e_rs = _make_spmm(NEP, PT, True)
_spmm_v = _make_spmm(NVP, PT, False)
_spmm_v_rs = _make_spmm(NVP, PT, True)
_spmm_s = _make_spmm(NEP, PTS, False)
_att = _make_att(PT)


def _pc(body, out_shapes, *args):
    return pl.pallas_call(
        body,
        out_shape=[jax.ShapeDtypeStruct(s, jnp.float32) for s in out_shapes],
    )(*args)


def _k1_body(h, w1, b1, w2, b2, w1a, hm, a1):
    hmv = jnp.maximum(h[...] @ w1[...] + b1[...], 0.0) @ w2[...] + b2[...]
    hm[...] = hmv
    a1[...] = hmv @ w1a[...]


def _k2_body(p, deg, w1b, b1, out):
    su = p[0] + p[1]
    ht = su * deg[...]
    out[...] = ht @ w1b[...] + b1[...]


def _k3_body(x, o):
    v = x[...]
    m = jnp.max(v)
    y = jnp.exp(v - m)
    flat = (lax.broadcasted_iota(jnp.int32, v.shape, 0) * v.shape[1]
            + lax.broadcasted_iota(jnp.int32, v.shape, 1))
    o[...] = jnp.where(flat < NNZ, y, 0.0)


def _k4_body(p, rs, out):
    su = p[0] + p[1]
    rsum = jnp.sum(rs[...], axis=0)[:, None] + 1e-20
    out[...] = su / rsum


def _bn(x, g, b):
    mu = jnp.mean(x, 0)
    var = jnp.mean((x - mu) ** 2, 0)
    return (x - mu) / jnp.sqrt(var + 1e-5) * g + b


def _k5_body(p, bng, bnb, mw1, mb1, mw2, mb2, w1a, hn1, a2):
    su = p[0] + p[1]
    hn_raw = su[:N_E, :]
    x = jnp.tanh(_bn(hn_raw, bng[...], bnb[...]))
    hn = jnp.maximum(x @ mw1[...] + mb1[...], 0.0) @ mw2[...] + mb2[...]
    hn1[...] = hn
    a2[...] = hn @ w1a[...]


def _k8_body(p, rs, h, bng, bnb, wiz, whz, wir, whr, win, whn, biz, bhz, bir,
             bhr, bin_, bhn, out):
    su = (p[0] + p[1])[:N_V, :]
    rsum = jnp.sum(rs[...], axis=0)[:N_V, None] + 1e-20
    hv = jnp.tanh(_bn(su / rsum, bng[...], bnb[...]))
    hh = h[...]
    z = jax.nn.sigmoid(hh @ wiz[...] + biz[...] + hv @ whz[...] + bhz[...])
    r = jax.nn.sigmoid(hh @ wir[...] + bir[...] + hv @ whr[...] + bhr[...])
    n = jnp.tanh(hh @ win[...] + bin_[...] + (r * hv) @ whn[...] + bhn[...])
    out[...] = (1.0 - z) * n + z * hv


def kernel(h, incident_nodes, incident_edges, incident_values, degree_v_values,
           degree_e_values, sent_index, sent_values, layer, params):
    p = params
    f32 = jnp.float32

    zpad = NNZ_PAD - NNZ
    inod = jnp.concatenate([incident_nodes.astype(jnp.int32),
                            jnp.zeros((zpad,), jnp.int32)])
    iedg = jnp.concatenate([incident_edges.astype(jnp.int32),
                            jnp.zeros((zpad,), jnp.int32)])
    ival = jnp.concatenate([incident_values, jnp.zeros((zpad,), f32)])
    szpad = NNZS_PAD - NNZS
    sdst = jnp.concatenate([sent_index[0].astype(jnp.int32),
                            jnp.zeros((szpad,), jnp.int32)])
    ssrc = jnp.concatenate([sent_index[1].astype(jnp.int32),
                            jnp.zeros((szpad,), jnp.int32)])
    sval = jnp.concatenate([sent_values, jnp.zeros((szpad,), f32)])
    deg_e = jnp.pad(degree_e_values, (0, NEP - N_E)).reshape(NEP, 1)
    deg_v = jnp.pad(degree_v_values, (0, NVP - N_V)).reshape(NVP, 1)
    att1_w1a = p['att1_W1'][:D]
    att1_w1b = p['att1_W1'][D:]
    att1_b1 = p['att1_b1'].reshape(1, DA)
    att1_w2 = p['att1_W2'].reshape(DA)
    att2_w1a = p['att2_W1'][:D]
    att2_w1b = p['att2_W1'][D:]
    att2_b1 = p['att2_b1'].reshape(1, DA)
    att2_w2 = p['att2_W2'].reshape(DA)

    hm, a1 = _pc(_k1_body, [(N_V, D), (N_V, DA)],
                 h, p['mlp1_W1'], p['mlp1_b1'].reshape(1, D), p['mlp1_W2'],
                 p['mlp1_b2'].reshape(1, D), att1_w1a)
    ht_p = _spmm_e(hm, inod, iedg, ival)
    b1t = _pc(_k2_body, [(NEP, DA)], ht_p, deg_e, att1_w1b, att1_b1)[0]
    att_raw = _att(a1, inod, b1t, iedg, att1_w2)
    att_w = _pc(_k3_body, [(NNZ_PAD // D, D)],
                att_raw.reshape(NNZ_PAD // D, D))[0].reshape(NNZ_PAD)
    pooled_p, rs1_p = _spmm_e_rs(hm, inod, iedg, att_w)
    he = _pc(_k4_body, [(NEP, D)], pooled_p, rs1_p)[0]

    hn_p = _spmm_s(he, ssrc, sdst, sval)
    hn, a2 = _pc(_k5_body, [(N_E, D), (N_E, DA)],
                 hn_p, p['bn2_g'].reshape(1, D), p['bn2_b'].reshape(1, D),
                 p['mlp2_W1'], p['mlp2_b1'].reshape(1, D), p['mlp2_W2'],
                 p['mlp2_b2'].reshape(1, D), att2_w1a)

    ht2_p = _spmm_v(hn, iedg, inod, ival)
    b2t = _pc(_k2_body, [(NVP, DA)], ht2_p, deg_v, att2_w1b, att2_b1)[0]
    att2_raw = _att(a2, iedg, b2t, inod, att2_w2)
    att2_w = _pc(_k3_body, [(NNZ_PAD // D, D)],
                 att2_raw.reshape(NNZ_PAD // D, D))[0].reshape(NNZ_PAD)
    pooled2_p, rs2_p = _spmm_v_rs(hn, iedg, inod, att2_w)

    out = _pc(_k8_body, [(N_V, D)],
              pooled2_p, rs2_p, h, p['bn1_g'].reshape(1, D),
              p['bn1_b'].reshape(1, D),
              p['gru_Wiz'], p['gru_Whz'], p['gru_Wir'], p['gru_Whr'],
              p['gru_Win'], p['gru_Whn'], p['gru_biz'].reshape(1, D),
              p['gru_bhz'].reshape(1, D), p['gru_bir'].reshape(1, D),
              p['gru_bhr'].reshape(1, D), p['gru_bin'].reshape(1, D),
              p['gru_bhn'].reshape(1, D))[0]
    return out


# speedup vs baseline: 1.7667x; 1.2405x over previous
import functools

import jax
import jax.numpy as jnp
from jax import lax
from jax.experimental import pallas as pl
from jax.experimental.pallas import tpu as pltpu
from jax.experimental.pallas import tpu_sc as plsc

NC, NS, L = 2, 16, 16
NW = NC * NS
D = 128
DA = 256

N_V, N_E = 10000, 5000
NNZ, NNZS = 320000, 80000
PT = 10240
NNZ_PAD = NW * PT
PTS = 3072
NNZS_PAD = NW * PTS
NEP = 5120
NVP = 10240
T = 128

_MESH = plsc.VectorSubcoreMesh(core_axis_name="c", subcore_axis_name="s",
                               num_cores=NC, num_subcores=NS)

_GATHER_DN = lax.GatherDimensionNumbers(
    offset_dims=(), collapsed_slice_dims=(0,), start_index_map=(0,))


def _bcast_lane(vec, i):
    idx = jnp.full((L, 1), i, jnp.int32)
    return lax.gather(vec, idx, _GATHER_DN, (1,),
                      mode=lax.GatherScatterMode.PROMISE_IN_BOUNDS)


def _lane_reduce_sum(v):
    lane_iota = lax.iota(jnp.int32, L)
    for sh in (8, 4, 2, 1):
        idx = (lane_iota ^ sh).reshape(L, 1)
        v = v + lax.gather(v, idx, _GATHER_DN, (1,),
                           mode=lax.GatherScatterMode.PROMISE_IN_BOUNDS)
    return v


def _make_spmm(ndst_pad, per_tile, with_rowsum):
    nb = per_tile // T
    if nb <= 24:
        CH = nb
    else:
        CH = 8
    nch = nb // CH
    assert nb % CH == 0 and CH % 2 == 0 and (nch == 1 or nch % 2 == 0)
    nib = 1 if nch == 1 else 2
    rsn = 5120 if ndst_pad <= 5120 else 10112
    zrows = ndst_pad // NS

    outs = jax.ShapeDtypeStruct((NC, ndst_pad, D), jnp.float32)
    scratch = [
        pltpu.VMEM((nib, CH, T), jnp.int32),
        pltpu.VMEM((nib, CH, T), jnp.int32),
        pltpu.VMEM((nib, CH, T), jnp.float32),
        pltpu.VMEM((T, D), jnp.float32),
        pltpu.VMEM((T, D), jnp.float32),
        pltpu.VMEM_SHARED((ndst_pad, D), jnp.float32),
        pltpu.SemaphoreType.DMA,
        pltpu.SemaphoreType.DMA,
        pltpu.SemaphoreType.DMA,
        pltpu.SemaphoreType.DMA,
        pltpu.SemaphoreType.DMA,
        pltpu.SemaphoreType.DMA,
    ]
    if with_rowsum:
        outs = (outs, jax.ShapeDtypeStruct((NW, rsn), jnp.float32))
        scratch.append(pltpu.VMEM((rsn,), jnp.float32))

    @functools.partial(pl.kernel, out_type=outs, mesh=_MESH,
                       scratch_types=scratch)
    def spmm(table, src_idx, dst_idx, w, *rest):
        if with_rowsum:
            (out, out_rs, sidx, didx, wbuf, rows0, rows1, acc,
             gs0, gs1, ss0, ss1, is0, is1, rs_v) = rest
        else:
            (out, sidx, didx, wbuf, rows0, rows1, acc,
             gs0, gs1, ss0, ss1, is0, is1) = rest
            out_rs = rs_v = None
        rows = (rows0, rows1)
        gsem = (gs0, gs1)
        ssem = (ss0, ss1)
        isem = (is0, is1)
        c = lax.axis_index("c")
        s = lax.axis_index("s")
        wid = c * NS + s
        lane_iota = lax.iota(jnp.int32, L)
        woff = pl.multiple_of(wid * nb, 8)

        def idx_copies(cn, p):
            co = pl.multiple_of(woff + cn * CH, 8)
            return (
                pltpu.make_async_copy(src_idx.at[pl.ds(co, CH)], sidx.at[p],
                                      isem[p]),
                pltpu.make_async_copy(dst_idx.at[pl.ds(co, CH)], didx.at[p],
                                      isem[p]),
                pltpu.make_async_copy(w.at[pl.ds(co, CH)], wbuf.at[p],
                                      isem[p]),
            )

        def idx_start(cn, p):
            for cp in idx_copies(cn, p):
                cp.start()

        def idx_wait(cn, p):
            for cp in idx_copies(cn, p):
                cp.wait()

        idx_start(0, 0)

        def zbody(i, carry):
            for cc in range(D // L):
                rows0[i, pl.ds(cc * L, L)] = jnp.zeros((L,), jnp.float32)
            return carry
        lax.fori_loop(0, T, zbody, 0)
        if with_rowsum:
            def zrs(i, carry):
                rs_v[pl.ds(i * L, L)] = jnp.zeros((L,), jnp.float32)
                return carry
            lax.fori_loop(0, rsn // L, zrs, 0)

        off = pl.multiple_of(s * zrows, 8)
        done = 0
        while done < zrows:
            ch = min(zrows - done, T)
            pltpu.sync_copy(rows0.at[pl.ds(0, ch)],
                            acc.at[pl.ds(off + done, ch)])
            done += ch
        plsc.subcore_barrier()

        idx_wait(0, 0)
        pltpu.async_copy(table.at[sidx.at[0, 0]], rows0, gs0)

        def batch(sb, pc, k2, k, last_in_chunk, chunk):
            cur, oth = rows[k], rows[1 - k]
            pltpu.make_async_copy(table.at[sidx.at[pc, 0]], cur,
                                  gsem[k]).wait()

            @pl.when(sb >= 1)
            def _():
                pltpu.make_async_copy(oth, acc.at[didx.at[pc, 0]],
                                      ssem[1 - k]).wait()
            if nch > 1:
                @pl.when((k2 == 0) & (chunk + 1 < nch))
                def _():
                    idx_start(chunk + 1, 1 - pc)
            @pl.when((sb + 1 < nb) & jnp.logical_not(last_in_chunk))
            def _():
                pltpu.async_copy(table.at[sidx.at[pc, k2 + 1]], oth,
                                 gsem[1 - k])
            if nch > 1:
                @pl.when(last_in_chunk & (chunk + 1 < nch))
                def _():
                    idx_wait(chunk + 1, 1 - pc)
                    pltpu.async_copy(table.at[sidx.at[1 - pc, 0]], oth,
                                     gsem[1 - k])
            def scale_group(g, carry):
                wv = wbuf[pc, k2, pl.ds(g * L, L)]
                if with_rowsum:
                    dv = didx[pc, k2, pl.ds(g * L, L)]
                for i in range(L):
                    rr = g * L + i
                    wb = _bcast_lane(wv, i)
                    for cc in range(D // L):
                        sl = pl.ds(cc * L, L)
                        cur[rr, sl] = cur[rr, sl] * wb
                    if with_rowsum:
                        e = dv[i]
                        win = rs_v[pl.ds(e, L)]
                        rs_v[pl.ds(e, L)] = win + jnp.where(
                            lane_iota == 0, wv[i], 0.0)
                return carry
            lax.fori_loop(0, T // L, scale_group, 0)
            pltpu.async_copy(cur, acc.at[didx.at[pc, k2]], ssem[k], add=True)

        if nch == 1:
            def rnd(r2, carry):
                for k in range(2):
                    k2 = r2 * 2 + k
                    batch(k2, 0, k2, k, jnp.bool_(False), 0)
                return carry
            lax.fori_loop(0, CH // 2, rnd, 0)
        else:
            def sc_rnd(c2, carry):
                for pc in range(2):
                    chunk = c2 * 2 + pc

                    def rnd(r2, carry2):
                        for k in range(2):
                            k2 = r2 * 2 + k
                            sb = chunk * CH + k2
                            batch(sb, pc, k2, k, k2 == CH - 1, chunk)
                        return carry2
                    lax.fori_loop(0, CH // 2, rnd, 0)
                return carry
            lax.fori_loop(0, nch // 2, sc_rnd, 0)

        pltpu.make_async_copy(rows1, acc.at[didx.at[0, 0]], ss1).wait()
        plsc.subcore_barrier()

        done = 0
        while done < zrows:
            ch = min(zrows - done, T)
            pltpu.sync_copy(acc.at[pl.ds(off + done, ch)],
                            rows0.at[pl.ds(0, ch)])
            pltpu.sync_copy(rows0.at[pl.ds(0, ch)],
                            out.at[c, pl.ds(off + done, ch)])
            done += ch
        if with_rowsum:
            pltpu.sync_copy(rs_v, out_rs.at[wid])

    return spmm


TA = 64


def _make_att(per_tile):
    nb = per_tile // TA
    assert nb % 2 == 0

    @functools.partial(
        pl.kernel,
        out_type=jax.ShapeDtypeStruct((NW * nb, TA), jnp.float32),
        mesh=_MESH,
        scratch_types=[
            pltpu.VMEM((nb, TA), jnp.int32),
            pltpu.VMEM((nb, TA), jnp.int32),
            pltpu.VMEM((TA, DA), jnp.float32),
            pltpu.VMEM((TA, DA), jnp.float32),
            pltpu.VMEM((TA, DA), jnp.float32),
            pltpu.VMEM((TA, DA), jnp.float32),
            pltpu.VMEM((DA,), jnp.float32),
            pltpu.VMEM((nb, TA), jnp.float32),
            pltpu.SemaphoreType.DMA,
            pltpu.SemaphoreType.DMA,
            pltpu.SemaphoreType.DMA,
            pltpu.SemaphoreType.DMA,
        ],
    )
    def att(tblA, idxA, tblB, idxB, w2, out, aidx, bidx, rA0, rA1, rB0, rB1,
            w2_v, attbuf, gsA0, gsA1, gsB0, gsB1):
        rowsA = (rA0, rA1)
        rowsB = (rB0, rB1)
        gsA = (gsA0, gsA1)
        gsB = (gsB0, gsB1)
        c = lax.axis_index("c")
        s = lax.axis_index("s")
        wid = c * NS + s
        woff = pl.multiple_of(wid * nb, 8)
        pltpu.sync_copy(idxA.at[pl.ds(woff, nb)], aidx)
        pltpu.sync_copy(idxB.at[pl.ds(woff, nb)], bidx)
        pltpu.sync_copy(w2, w2_v)
        lane_iota = lax.iota(jnp.int32, L)

        pltpu.async_copy(tblA.at[aidx.at[0]], rA0, gsA0)
        pltpu.async_copy(tblB.at[bidx.at[0]], rB0, gsB0)

        def rnd(r, carry):
            for k in range(2):
                sb = r * 2 + k
                curA, othA = rowsA[k], rowsA[1 - k]
                curB, othB = rowsB[k], rowsB[1 - k]
                pltpu.make_async_copy(tblA.at[aidx.at[sb]], curA,
                                      gsA[k]).wait()
                pltpu.make_async_copy(tblB.at[bidx.at[sb]], curB,
                                      gsB[k]).wait()

                @pl.when(sb + 1 < nb)
                def _():
                    pltpu.async_copy(tblA.at[aidx.at[sb + 1]], othA,
                                     gsA[1 - k])
                    pltpu.async_copy(tblB.at[bidx.at[sb + 1]], othB,
                                     gsB[1 - k])
                w2c = [w2_v[pl.ds(cc * L, L)] for cc in range(DA // L)]

                def group(g, carry2):
                    accv = jnp.zeros((L,), jnp.float32)
                    for i in range(L):
                        rr = g * L + i
                        acc = jnp.zeros((L,), jnp.float32)
                        for cc in range(DA // L):
                            sl = pl.ds(cc * L, L)
                            t = curA[rr, sl] + curB[rr, sl]
                            m = jnp.maximum(t, 0.01 * t)
                            acc = acc + m * w2c[cc]
                        red = _lane_reduce_sum(acc)
                        accv = jnp.where(lane_iota == i, red, accv)
                    attbuf[sb, pl.ds(g * L, L)] = accv
                    return carry2
                lax.fori_loop(0, TA // L, group, 0)
            return carry
        lax.fori_loop(0, nb // 2, rnd, 0)
        pltpu.sync_copy(attbuf, out.at[pl.ds(woff, nb)])

    return att


_spmm_e = _make_spmm(NEP, PT, False)
_spmm_e_rs = _make_spmm(NEP, PT, True)
_spmm_v = _make_spmm(NVP, PT, False)
_spmm_v_rs = _make_spmm(NVP, PT, True)
_spmm_s = _make_spmm(NEP, PTS, False)
_att = _make_att(PT)


def _pc(body, out_shapes, *args):
    return pl.pallas_call(
        body,
        out_shape=[jax.ShapeDtypeStruct(s, jnp.float32) for s in out_shapes],
    )(*args)


def _k1_body(h, w1, b1, w2, b2, w1a, hm, a1):
    hmv = jnp.maximum(h[...] @ w1[...] + b1[...], 0.0) @ w2[...] + b2[...]
    hm[...] = hmv
    a1[...] = hmv @ w1a[...]


def _k2_body(p, deg, w1b, b1, out):
    su = p[0] + p[1]
    ht = su * deg[...]
    out[...] = ht @ w1b[...] + b1[...]


def _k3_body(x, o):
    v = x[...]
    m = jnp.max(v)
    y = jnp.exp(v - m)
    flat = (lax.broadcasted_iota(jnp.int32, v.shape, 0) * v.shape[1]
            + lax.broadcasted_iota(jnp.int32, v.shape, 1))
    o[...] = jnp.where(flat < NNZ, y, 0.0)


def _k4_body(p, rs, out):
    su = p[0] + p[1]
    rsum = jnp.sum(rs[...], axis=0)[:, None] + 1e-20
    out[...] = su / rsum


def _bn(x, g, b):
    mu = jnp.mean(x, 0)
    var = jnp.mean((x - mu) ** 2, 0)
    return (x - mu) / jnp.sqrt(var + 1e-5) * g + b


def _k5_body(p, bng, bnb, mw1, mb1, mw2, mb2, w1a, hn1, a2):
    su = p[0] + p[1]
    hn_raw = su[:N_E, :]
    x = jnp.tanh(_bn(hn_raw, bng[...], bnb[...]))
    hn = jnp.maximum(x @ mw1[...] + mb1[...], 0.0) @ mw2[...] + mb2[...]
    hn1[...] = hn
    a2[...] = hn @ w1a[...]


def _k8_body(p, rs, h, bng, bnb, wiz, whz, wir, whr, win, whn, biz, bhz, bir,
             bhr, bin_, bhn, out):
    su = (p[0] + p[1])[:N_V, :]
    rsum = jnp.sum(rs[...], axis=0)[:N_V, None] + 1e-20
    hv = jnp.tanh(_bn(su / rsum, bng[...], bnb[...]))
    hh = h[...]
    z = jax.nn.sigmoid(hh @ wiz[...] + biz[...] + hv @ whz[...] + bhz[...])
    r = jax.nn.sigmoid(hh @ wir[...] + bir[...] + hv @ whr[...] + bhr[...])
    n = jnp.tanh(hh @ win[...] + bin_[...] + (r * hv) @ whn[...] + bhn[...])
    out[...] = (1.0 - z) * n + z * hv


def kernel(h, incident_nodes, incident_edges, incident_values, degree_v_values,
           degree_e_values, sent_index, sent_values, layer, params):
    p = params
    f32 = jnp.float32

    zpad = NNZ_PAD - NNZ
    nbi = PT // T
    nba = PT // TA
    inod = jnp.concatenate([incident_nodes.astype(jnp.int32),
                            jnp.zeros((zpad,), jnp.int32)])
    iedg = jnp.concatenate([incident_edges.astype(jnp.int32),
                            jnp.zeros((zpad,), jnp.int32)])
    ival = jnp.concatenate([incident_values, jnp.zeros((zpad,), f32)])
    inod2 = inod.reshape(NW * nbi, T)
    iedg2 = iedg.reshape(NW * nbi, T)
    ival2 = ival.reshape(NW * nbi, T)
    inodA = inod.reshape(NW * nba, TA)
    iedgA = iedg.reshape(NW * nba, TA)
    szpad = NNZS_PAD - NNZS
    nbs = PTS // T
    sdst = jnp.concatenate([sent_index[0].astype(jnp.int32),
                            jnp.zeros((szpad,), jnp.int32)]).reshape(
                                NW * nbs, T)
    ssrc = jnp.concatenate([sent_index[1].astype(jnp.int32),
                            jnp.zeros((szpad,), jnp.int32)]).reshape(
                                NW * nbs, T)
    sval = jnp.concatenate([sent_values, jnp.zeros((szpad,), f32)]).reshape(
        NW * nbs, T)
    deg_e = jnp.pad(degree_e_values, (0, NEP - N_E)).reshape(NEP, 1)
    deg_v = jnp.pad(degree_v_values, (0, NVP - N_V)).reshape(NVP, 1)
    att1_w1a = p['att1_W1'][:D]
    att1_w1b = p['att1_W1'][D:]
    att1_b1 = p['att1_b1'].reshape(1, DA)
    att1_w2 = p['att1_W2'].reshape(DA)
    att2_w1a = p['att2_W1'][:D]
    att2_w1b = p['att2_W1'][D:]
    att2_b1 = p['att2_b1'].reshape(1, DA)
    att2_w2 = p['att2_W2'].reshape(DA)

    hm, a1 = _pc(_k1_body, [(N_V, D), (N_V, DA)],
                 h, p['mlp1_W1'], p['mlp1_b1'].reshape(1, D), p['mlp1_W2'],
                 p['mlp1_b2'].reshape(1, D), att1_w1a)
    ht_p = _spmm_e(hm, inod2, iedg2, ival2)
    b1t = _pc(_k2_body, [(NEP, DA)], ht_p, deg_e, att1_w1b, att1_b1)[0]
    att_raw = _att(a1, inodA, b1t, iedgA, att1_w2)
    att_w = _pc(_k3_body, [(NNZ_PAD // D, D)],
                att_raw.reshape(NNZ_PAD // D, D))[0].reshape(NW * nbi, T)
    pooled_p, rs1_p = _spmm_e_rs(hm, inod2, iedg2, att_w)
    he = _pc(_k4_body, [(NEP, D)], pooled_p, rs1_p)[0]

    hn_p = _spmm_s(he, ssrc, sdst, sval)
    hn, a2 = _pc(_k5_body, [(N_E, D), (N_E, DA)],
                 hn_p, p['bn2_g'].reshape(1, D), p['bn2_b'].reshape(1, D),
                 p['mlp2_W1'], p['mlp2_b1'].reshape(1, D), p['mlp2_W2'],
                 p['mlp2_b2'].reshape(1, D), att2_w1a)

    ht2_p = _spmm_v(hn, iedg2, inod2, ival2)
    b2t = _pc(_k2_body, [(NVP, DA)], ht2_p, deg_v, att2_w1b, att2_b1)[0]
    att2_raw = _att(a2, iedgA, b2t, inodA, att2_w2)
    att2_w = _pc(_k3_body, [(NNZ_PAD // D, D)],
                 att2_raw.reshape(NNZ_PAD // D, D))[0].reshape(NW * nbi, T)
    pooled2_p, rs2_p = _spmm_v_rs(hn, iedg2, inod2, att2_w)

    out = _pc(_k8_body, [(N_V, D)],
              pooled2_p, rs2_p, h, p['bn1_g'].reshape(1, D),
              p['bn1_b'].reshape(1, D),
              p['gru_Wiz'], p['gru_Whz'], p['gru_Wir'], p['gru_Whr'],
              p['gru_Win'], p['gru_Whn'], p['gru_biz'].reshape(1, D),
              p['gru_bhz'].reshape(1, D), p['gru_bir'].reshape(1, D),
              p['gru_bhr'].reshape(1, D), p['gru_bin'].reshape(1, D),
              p['gru_bhn'].reshape(1, D))[0]
    return out
